# baseline (speedup 1.0000x reference)
import functools
import math

import jax
import jax.numpy as jnp
from jax import lax
from jax.experimental import pallas as pl
from jax.experimental.pallas import tpu as pltpu
from jax.experimental.pallas import tpu_sc as plsc

D_MODEL = 256
CARD_START, CARD_END = 1, 8
NCARD = CARD_END - CARD_START
NC, NS, L = 2, 16, 16
NW = NC * NS
CHUNK = 64
ZERO_ROW = 208


def _table_body(rank_ref, suit_ref, street_ref, out_ref):
    i = lax.broadcasted_iota(jnp.int32, (256, 1), 0)
    oh_r = (i // 16 == lax.broadcasted_iota(jnp.int32, (256, 16), 1)).astype(jnp.float32)
    oh_s = ((i // 4) % 4 == lax.broadcasted_iota(jnp.int32, (256, 8), 1)).astype(jnp.float32)
    oh_t = (i % 4 == lax.broadcasted_iota(jnp.int32, (256, 8), 1)).astype(jnp.float32)
    hi = jax.lax.Precision.HIGHEST
    acc = jnp.dot(oh_r, rank_ref[...], preferred_element_type=jnp.float32, precision=hi)
    acc += jnp.dot(oh_s, suit_ref[...], preferred_element_type=jnp.float32, precision=hi)
    acc += jnp.dot(oh_t, street_ref[...], preferred_element_type=jnp.float32, precision=hi)
    out_ref[...] = jnp.where(i < ZERO_ROW, acc, 0.0)


def _build_table(rank_emb, suit_emb, street_emb):
    rank_p = jnp.pad(rank_emb, ((0, 3), (0, 0)))
    suit_p = jnp.pad(suit_emb, ((0, 4), (0, 0)))
    street_p = jnp.pad(street_emb, ((0, 4), (0, 0)))
    return pl.pallas_call(
        _table_body,
        out_shape=jax.ShapeDtypeStruct((256, D_MODEL), jnp.float32),
    )(rank_p, suit_p, street_p)


NBUF = 6
AHEAD = 3


def _sc_body(nchunks, row0, tok_hbm, st_hbm, r_hbm, s_hbm, table_hbm, out_hbm,
             tok_v, st_v, r_v, s_v, fidx_v, rows_v, gat_sem, wr_sem):
    wid = lax.axis_index("s") * NC + lax.axis_index("c")
    n = nchunks * CHUNK
    islab = wid * n
    slab = row0 + islab

    sl_all = pl.ds(islab, n)
    pltpu.sync_copy(tok_hbm.at[sl_all], tok_v)
    pltpu.sync_copy(st_hbm.at[sl_all], st_v)
    pltpu.sync_copy(r_hbm.at[sl_all], r_v)
    pltpu.sync_copy(s_hbm.at[sl_all], s_v)

    def fuse(i, _):
        vsl = pl.ds(i * L, L)
        f = r_v[vsl] * 16 + s_v[vsl] * 4 + st_v[vsl]
        fidx_v[vsl] = jnp.where(tok_v[vsl] >= 0, f, ZERO_ROW)
        return 0

    lax.fori_loop(0, n // L, fuse, 0)

    def gather_desc(c, slot):
        return pltpu.make_async_copy(
            table_hbm.at[fidx_v.at[pl.ds(c * CHUNK, CHUNK)]],
            rows_v.at[slot], gat_sem.at[slot])

    def write_desc(c, slot):
        return pltpu.make_async_copy(
            rows_v.at[slot], out_hbm.at[pl.ds(slab + c * CHUNK, CHUNK)],
            wr_sem.at[slot])

    for c in range(AHEAD):
        gather_desc(c, c % NBUF).start()
    for c in range(AHEAD):
        gather_desc(c, c % NBUF).wait()
        write_desc(c, c % NBUF).start()
        gather_desc(c + AHEAD, (c + AHEAD) % NBUF).start()

    def body(c, _):
        slot = lax.rem(c, NBUF)
        gather_desc(c, slot).wait()
        write_desc(c, slot).start()
        write_desc(c - AHEAD, lax.rem(c - AHEAD, NBUF)).wait()
        gather_desc(c + AHEAD, lax.rem(c + AHEAD, NBUF)).start()
        return 0

    lax.fori_loop(AHEAD, nchunks - AHEAD, body, 0)

    for c in range(nchunks - AHEAD, nchunks):
        gather_desc(c, c % NBUF).wait()
        write_desc(c, c % NBUF).start()
    for c in range(nchunks - 2 * AHEAD, nchunks):
        write_desc(c, c % NBUF).wait()


TC_ROWS = 4096
EMB_K = 32
SC_FRAC_NUM, SC_FRAC_DEN = 1, 8


def _tc_main_body(ppj, tok_ref, st_ref, r_ref, s_ref, emb_ref, buf_ref,
                  out_ref):
    del buf_ref
    col = pl.program_id(0) // ppj + 1
    li = lax.broadcasted_iota(jnp.int32, (TC_ROWS, 8), 1)
    pick = li == col

    def ex(ref):
        return jnp.sum(jnp.where(pick, ref[...], 0), axis=1, keepdims=True)

    tok_c, r_c, s_c, st_c = ex(tok_ref), ex(r_ref), ex(s_ref), ex(st_ref)
    k = lax.broadcasted_iota(jnp.int32, (TC_ROWS, EMB_K), 1)
    cond = (((k == r_c) | (k == s_c + 16) | (k == st_c + 24)) & (tok_c >= 0))
    oh = jnp.where(cond, 1.0, 0.0).astype(jnp.bfloat16)
    oh4 = jnp.concatenate([oh, oh, oh, jnp.zeros_like(oh)], axis=1)
    out_ref[...] = jnp.dot(oh4, emb_ref[...],
                           preferred_element_type=jnp.float32)


def kernel(token_ids, card_streets, card_ranks, card_suits,
           rank_emb, suit_emb, street_emb):
    B = token_ids.shape[0]
    n_rows = B * NCARD
    n_sc = n_rows * SC_FRAC_NUM // SC_FRAC_DEN
    n_sc -= n_sc % (NW * CHUNK * TC_ROWS // math.gcd(NW * CHUNK, TC_ROWS))
    n_tc = n_rows - n_sc
    assert n_tc % TC_ROWS == 0 and n_sc % (NW * CHUNK) == 0
    nchunks = n_sc // (NW * CHUNK)

    assert n_sc <= B and B % TC_ROWS == 0
    b0 = B - n_sc
    tokS = token_ids[b0:, CARD_END - 1].astype(jnp.int32)
    stS = card_streets[b0:, CARD_END - 1].astype(jnp.int32)
    rS = card_ranks[b0:, CARD_END - 1].astype(jnp.int32)
    sS = card_suits[b0:, CARD_END - 1].astype(jnp.int32)
    tok8 = token_ids[:, 0:8].astype(jnp.int32)
    st8 = card_streets[:, 0:8].astype(jnp.int32)
    r8 = card_ranks[:, 0:8].astype(jnp.int32)
    s8 = card_suits[:, 0:8].astype(jnp.int32)

    ctable = _build_table(rank_emb, suit_emb, street_emb)

    sc = pl.kernel(
        functools.partial(_sc_body, nchunks, n_tc),
        out_type=jax.ShapeDtypeStruct((n_rows, D_MODEL), jnp.float32),
        mesh=plsc.VectorSubcoreMesh(core_axis_name="c", subcore_axis_name="s"),
        scratch_types=[
            pltpu.VMEM((nchunks * CHUNK,), jnp.int32),
            pltpu.VMEM((nchunks * CHUNK,), jnp.int32),
            pltpu.VMEM((nchunks * CHUNK,), jnp.int32),
            pltpu.VMEM((nchunks * CHUNK,), jnp.int32),
            pltpu.VMEM((nchunks * CHUNK,), jnp.int32),
            pltpu.VMEM((NBUF, CHUNK, D_MODEL), jnp.float32),
            pltpu.SemaphoreType.DMA((NBUF,)),
            pltpu.SemaphoreType.DMA((NBUF,)),
        ],
    )
    y = sc(tokS, stS, rS, sS, ctable)

    nblk = n_tc // TC_ROWS
    ppj = B // TC_ROWS
    stacked = jnp.concatenate([
        jnp.pad(rank_emb, ((0, 3), (0, 0))),
        jnp.pad(suit_emb, ((0, 4), (0, 0))),
        jnp.pad(street_emb, ((0, EMB_K - 28), (0, 0))),
    ], axis=0)
    def trunc16(x):
        u = lax.bitcast_convert_type(x, jnp.uint32)
        return lax.bitcast_convert_type(u & jnp.uint32(0xFFFF0000), jnp.float32)
    e1f = trunc16(stacked)
    r1 = stacked - e1f
    e2f = trunc16(r1)
    r2 = r1 - e2f
    e123 = jnp.concatenate([e1f.astype(jnp.bfloat16),
                            e2f.astype(jnp.bfloat16),
                            r2.astype(jnp.bfloat16),
                            jnp.zeros((EMB_K, D_MODEL), jnp.bfloat16)], axis=0)
    idx_spec = pl.BlockSpec((TC_ROWS, 8), lambda i: (lax.rem(i, ppj), 0))
    y = pl.pallas_call(
        functools.partial(_tc_main_body, ppj),
        grid=(nblk,),
        in_specs=[idx_spec, idx_spec, idx_spec, idx_spec,
                  pl.BlockSpec((4 * EMB_K, D_MODEL), lambda i: (0, 0)),
                  pl.BlockSpec(memory_space=pl.ANY)],
        out_specs=pl.BlockSpec((TC_ROWS, D_MODEL), lambda i: (i, 0)),
        out_shape=jax.ShapeDtypeStruct((n_rows, D_MODEL), jnp.float32),
        input_output_aliases={5: 0},
    )(tok8, st8, r8, s8, e123, y)

    return jnp.transpose(y.reshape(NCARD, B, D_MODEL), (1, 0, 2))


# speedup vs baseline: 2.3413x; 2.3413x over previous
import functools
import math

import jax
import jax.numpy as jnp
from jax import lax
from jax.experimental import pallas as pl
from jax.experimental.pallas import tpu as pltpu
from jax.experimental.pallas import tpu_sc as plsc

D_MODEL = 256
CARD_START, CARD_END = 1, 8
NCARD = CARD_END - CARD_START
NC, NS, L = 2, 16, 16
NW = NC * NS
CHUNK = 64
ZERO_ROW = 208


def _table_body(rank_ref, suit_ref, street_ref, out_ref):
    i = lax.broadcasted_iota(jnp.int32, (256, 1), 0)
    oh_r = (i // 16 == lax.broadcasted_iota(jnp.int32, (256, 16), 1)).astype(jnp.float32)
    oh_s = ((i // 4) % 4 == lax.broadcasted_iota(jnp.int32, (256, 8), 1)).astype(jnp.float32)
    oh_t = (i % 4 == lax.broadcasted_iota(jnp.int32, (256, 8), 1)).astype(jnp.float32)
    hi = jax.lax.Precision.HIGHEST
    acc = jnp.dot(oh_r, rank_ref[...], preferred_element_type=jnp.float32, precision=hi)
    acc += jnp.dot(oh_s, suit_ref[...], preferred_element_type=jnp.float32, precision=hi)
    acc += jnp.dot(oh_t, street_ref[...], preferred_element_type=jnp.float32, precision=hi)
    out_ref[...] = jnp.where(i < ZERO_ROW, acc, 0.0)


def _build_table(rank_emb, suit_emb, street_emb):
    rank_p = jnp.pad(rank_emb, ((0, 3), (0, 0)))
    suit_p = jnp.pad(suit_emb, ((0, 4), (0, 0)))
    street_p = jnp.pad(street_emb, ((0, 4), (0, 0)))
    return pl.pallas_call(
        _table_body,
        out_shape=jax.ShapeDtypeStruct((256, D_MODEL), jnp.float32),
    )(rank_p, suit_p, street_p)


NBUF = 6
AHEAD = 3


def _sc_body(nchunks, row0, tok_hbm, st_hbm, r_hbm, s_hbm, table_hbm, out_hbm,
             tok_v, st_v, r_v, s_v, fidx_v, rows_v, gat_sem, wr_sem):
    wid = lax.axis_index("s") * NC + lax.axis_index("c")
    n = nchunks * CHUNK
    slab = row0 + wid * n

    sl_all = pl.ds(slab, n)
    pltpu.sync_copy(tok_hbm.at[sl_all], tok_v)
    pltpu.sync_copy(st_hbm.at[sl_all], st_v)
    pltpu.sync_copy(r_hbm.at[sl_all], r_v)
    pltpu.sync_copy(s_hbm.at[sl_all], s_v)

    def fuse(i, _):
        vsl = pl.ds(i * L, L)
        f = r_v[vsl] * 16 + s_v[vsl] * 4 + st_v[vsl]
        fidx_v[vsl] = jnp.where(tok_v[vsl] >= 0, f, ZERO_ROW)
        return 0

    lax.fori_loop(0, n // L, fuse, 0)

    def gather_desc(c, slot):
        return pltpu.make_async_copy(
            table_hbm.at[fidx_v.at[pl.ds(c * CHUNK, CHUNK)]],
            rows_v.at[slot], gat_sem.at[slot])

    def write_desc(c, slot):
        return pltpu.make_async_copy(
            rows_v.at[slot], out_hbm.at[pl.ds(slab + c * CHUNK, CHUNK)],
            wr_sem.at[slot])

    for c in range(AHEAD):
        gather_desc(c, c % NBUF).start()
    for c in range(AHEAD):
        gather_desc(c, c % NBUF).wait()
        write_desc(c, c % NBUF).start()
        gather_desc(c + AHEAD, (c + AHEAD) % NBUF).start()

    def body(c, _):
        slot = lax.rem(c, NBUF)
        gather_desc(c, slot).wait()
        write_desc(c, slot).start()
        write_desc(c - AHEAD, lax.rem(c - AHEAD, NBUF)).wait()
        gather_desc(c + AHEAD, lax.rem(c + AHEAD, NBUF)).start()
        return 0

    lax.fori_loop(AHEAD, nchunks - AHEAD, body, 0)

    for c in range(nchunks - AHEAD, nchunks):
        gather_desc(c, c % NBUF).wait()
        write_desc(c, c % NBUF).start()
    for c in range(nchunks - 2 * AHEAD, nchunks):
        write_desc(c, c % NBUF).wait()


TC_ROWS = 4096
EMB_K = 32
SC_FRAC_NUM, SC_FRAC_DEN = 1, 8


def _tc_main_body(tok_ref, st_ref, r_ref, s_ref, emb_ref, buf_ref, out_ref):
    del buf_ref
    j = lax.broadcasted_iota(jnp.int32, (EMB_K, 128), 0)
    for q in range(TC_ROWS // 128):
        qs = pl.ds(q, 1)
        cond = (((j == r_ref[0, qs, :]) | (j == s_ref[0, qs, :] + 16)
                 | (j == st_ref[0, qs, :] + 24))
                & (tok_ref[0, qs, :] >= 0))
        oh = jnp.where(cond, 1.0, 0.0).astype(jnp.bfloat16)
        oh3 = jnp.concatenate([oh, oh, oh], axis=0)
        out_ref[pl.ds(q * 128, 128), :] = lax.dot_general(
            oh3, emb_ref[...], dimension_numbers=(((0,), (0,)), ((), ())),
            preferred_element_type=jnp.float32)


def kernel(token_ids, card_streets, card_ranks, card_suits,
           rank_emb, suit_emb, street_emb):
    B = token_ids.shape[0]
    n_rows = B * NCARD
    n_sc = n_rows * SC_FRAC_NUM // SC_FRAC_DEN
    n_sc -= n_sc % (NW * CHUNK * TC_ROWS // math.gcd(NW * CHUNK, TC_ROWS))
    n_tc = n_rows - n_sc
    assert n_tc % TC_ROWS == 0 and n_sc % (NW * CHUNK) == 0
    nchunks = n_sc // (NW * CHUNK)

    tok7 = token_ids[:, CARD_START:CARD_END].T.reshape(-1).astype(jnp.int32)
    st7 = card_streets[:, CARD_START:CARD_END].T.reshape(-1).astype(jnp.int32)
    r7 = card_ranks[:, CARD_START:CARD_END].T.reshape(-1).astype(jnp.int32)
    s7 = card_suits[:, CARD_START:CARD_END].T.reshape(-1).astype(jnp.int32)

    ctable = _build_table(rank_emb, suit_emb, street_emb)

    sc = pl.kernel(
        functools.partial(_sc_body, nchunks, n_tc),
        out_type=jax.ShapeDtypeStruct((n_rows, D_MODEL), jnp.float32),
        mesh=plsc.VectorSubcoreMesh(core_axis_name="c", subcore_axis_name="s"),
        scratch_types=[
            pltpu.VMEM((nchunks * CHUNK,), jnp.int32),
            pltpu.VMEM((nchunks * CHUNK,), jnp.int32),
            pltpu.VMEM((nchunks * CHUNK,), jnp.int32),
            pltpu.VMEM((nchunks * CHUNK,), jnp.int32),
            pltpu.VMEM((nchunks * CHUNK,), jnp.int32),
            pltpu.VMEM((NBUF, CHUNK, D_MODEL), jnp.float32),
            pltpu.SemaphoreType.DMA((NBUF,)),
            pltpu.SemaphoreType.DMA((NBUF,)),
        ],
    )
    y = sc(tok7, st7, r7, s7, ctable)

    nblk = n_tc // TC_ROWS
    blk3 = lambda a: a[:n_tc].reshape(nblk, TC_ROWS // 128, 128)
    stacked = jnp.concatenate([
        jnp.pad(rank_emb, ((0, 3), (0, 0))),
        jnp.pad(suit_emb, ((0, 4), (0, 0))),
        jnp.pad(street_emb, ((0, EMB_K - 28), (0, 0))),
    ], axis=0)
    def trunc16(x):
        u = lax.bitcast_convert_type(x, jnp.uint32)
        return lax.bitcast_convert_type(u & jnp.uint32(0xFFFF0000), jnp.float32)
    e1f = trunc16(stacked)
    r1 = stacked - e1f
    e2f = trunc16(r1)
    r2 = r1 - e2f
    e123 = jnp.concatenate([e1f.astype(jnp.bfloat16),
                            e2f.astype(jnp.bfloat16),
                            r2.astype(jnp.bfloat16)], axis=0)
    idx_spec = pl.BlockSpec((1, TC_ROWS // 128, 128), lambda i: (i, 0, 0))
    y = pl.pallas_call(
        _tc_main_body,
        grid=(nblk,),
        in_specs=[idx_spec, idx_spec, idx_spec, idx_spec,
                  pl.BlockSpec((3 * EMB_K, D_MODEL), lambda i: (0, 0)),
                  pl.BlockSpec(memory_space=pl.ANY)],
        out_specs=pl.BlockSpec((TC_ROWS, D_MODEL), lambda i: (i, 0)),
        out_shape=jax.ShapeDtypeStruct((n_rows, D_MODEL), jnp.float32),
        input_output_aliases={5: 0},
    )(blk3(tok7), blk3(st7), blk3(r7), blk3(s7), e123, y)

    return jnp.transpose(y.reshape(NCARD, B, D_MODEL), (1, 0, 2))


# speedup vs baseline: 2.3682x; 1.0115x over previous
import functools
import math

import jax
import jax.numpy as jnp
from jax import lax
from jax.experimental import pallas as pl
from jax.experimental.pallas import tpu as pltpu
from jax.experimental.pallas import tpu_sc as plsc

D_MODEL = 256
CARD_START, CARD_END = 1, 8
NCARD = CARD_END - CARD_START
NC, NS, L = 2, 16, 16
NW = NC * NS
CHUNK = 64
ZERO_ROW = 208


def _table_body(rank_ref, suit_ref, street_ref, out_ref):
    i = lax.broadcasted_iota(jnp.int32, (256, 1), 0)
    oh_r = (i // 16 == lax.broadcasted_iota(jnp.int32, (256, 16), 1)).astype(jnp.float32)
    oh_s = ((i // 4) % 4 == lax.broadcasted_iota(jnp.int32, (256, 8), 1)).astype(jnp.float32)
    oh_t = (i % 4 == lax.broadcasted_iota(jnp.int32, (256, 8), 1)).astype(jnp.float32)
    hi = jax.lax.Precision.HIGHEST
    acc = jnp.dot(oh_r, rank_ref[...], preferred_element_type=jnp.float32, precision=hi)
    acc += jnp.dot(oh_s, suit_ref[...], preferred_element_type=jnp.float32, precision=hi)
    acc += jnp.dot(oh_t, street_ref[...], preferred_element_type=jnp.float32, precision=hi)
    out_ref[...] = jnp.where(i < ZERO_ROW, acc, 0.0)


def _build_table(rank_emb, suit_emb, street_emb):
    rank_p = jnp.pad(rank_emb, ((0, 3), (0, 0)))
    suit_p = jnp.pad(suit_emb, ((0, 4), (0, 0)))
    street_p = jnp.pad(street_emb, ((0, 4), (0, 0)))
    return pl.pallas_call(
        _table_body,
        out_shape=jax.ShapeDtypeStruct((256, D_MODEL), jnp.float32),
    )(rank_p, suit_p, street_p)


NBUF = 6
AHEAD = 3


def _sc_body(nchunks, row0, tok_hbm, st_hbm, r_hbm, s_hbm, table_hbm, out_hbm,
             tok_v, st_v, r_v, s_v, fidx_v, rows_v, gat_sem, wr_sem):
    wid = lax.axis_index("s") * NC + lax.axis_index("c")
    n = nchunks * CHUNK
    slab = row0 + wid * n

    sl_all = pl.ds(wid * n, n)
    pltpu.sync_copy(tok_hbm.at[sl_all], tok_v)
    pltpu.sync_copy(st_hbm.at[sl_all], st_v)
    pltpu.sync_copy(r_hbm.at[sl_all], r_v)
    pltpu.sync_copy(s_hbm.at[sl_all], s_v)

    def fuse(i, _):
        vsl = pl.ds(i * L, L)
        f = r_v[vsl] * 16 + s_v[vsl] * 4 + st_v[vsl]
        fidx_v[vsl] = jnp.where(tok_v[vsl] >= 0, f, ZERO_ROW)
        return 0

    lax.fori_loop(0, n // L, fuse, 0)

    def gather_desc(c, slot):
        return pltpu.make_async_copy(
            table_hbm.at[fidx_v.at[pl.ds(c * CHUNK, CHUNK)]],
            rows_v.at[slot], gat_sem.at[slot])

    def write_desc(c, slot):
        return pltpu.make_async_copy(
            rows_v.at[slot], out_hbm.at[pl.ds(slab + c * CHUNK, CHUNK)],
            wr_sem.at[slot])

    for c in range(AHEAD):
        gather_desc(c, c % NBUF).start()
    for c in range(AHEAD):
        gather_desc(c, c % NBUF).wait()
        write_desc(c, c % NBUF).start()
        gather_desc(c + AHEAD, (c + AHEAD) % NBUF).start()

    def body(c, _):
        slot = lax.rem(c, NBUF)
        gather_desc(c, slot).wait()
        write_desc(c, slot).start()
        write_desc(c - AHEAD, lax.rem(c - AHEAD, NBUF)).wait()
        gather_desc(c + AHEAD, lax.rem(c + AHEAD, NBUF)).start()
        return 0

    lax.fori_loop(AHEAD, nchunks - AHEAD, body, 0)

    for c in range(nchunks - AHEAD, nchunks):
        gather_desc(c, c % NBUF).wait()
        write_desc(c, c % NBUF).start()
    for c in range(nchunks - 2 * AHEAD, nchunks):
        write_desc(c, c % NBUF).wait()


TC_ROWS = 4096
EMB_K = 32
SC_FRAC_NUM, SC_FRAC_DEN = 1, 8


def _tc_main_body(tok_ref, st_ref, r_ref, s_ref, emb_ref, buf_ref, out_ref):
    del buf_ref
    j = lax.broadcasted_iota(jnp.int32, (EMB_K, 128), 0)
    for q in range(TC_ROWS // 128):
        qs = pl.ds(q, 1)
        cond = (((j == r_ref[0, qs, :]) | (j == s_ref[0, qs, :] + 16)
                 | (j == st_ref[0, qs, :] + 24))
                & (tok_ref[0, qs, :] >= 0))
        oh = jnp.where(cond, 1.0, 0.0).astype(jnp.bfloat16)
        oh3 = jnp.concatenate([oh, oh, oh], axis=0)
        out_ref[pl.ds(q * 128, 128), :] = lax.dot_general(
            oh3, emb_ref[...], dimension_numbers=(((0,), (0,)), ((), ())),
            preferred_element_type=jnp.float32)


def kernel(token_ids, card_streets, card_ranks, card_suits,
           rank_emb, suit_emb, street_emb):
    B = token_ids.shape[0]
    n_rows = B * NCARD
    n_sc = n_rows * SC_FRAC_NUM // SC_FRAC_DEN
    n_sc -= n_sc % (NW * CHUNK * TC_ROWS // math.gcd(NW * CHUNK, TC_ROWS))
    n_tc = n_rows - n_sc
    assert n_tc % TC_ROWS == 0 and n_sc % (NW * CHUNK) == 0
    nchunks = n_sc // (NW * CHUNK)

    tok7 = token_ids[:, CARD_START:CARD_END].T.reshape(-1).astype(jnp.int32)
    st7 = card_streets[:, CARD_START:CARD_END].T.reshape(-1).astype(jnp.int32)
    r7 = card_ranks[:, CARD_START:CARD_END].T.reshape(-1).astype(jnp.int32)
    s7 = card_suits[:, CARD_START:CARD_END].T.reshape(-1).astype(jnp.int32)

    assert n_sc <= B
    b0 = B - n_sc
    tokS = token_ids[b0:, CARD_END - 1].astype(jnp.int32)
    stS = card_streets[b0:, CARD_END - 1].astype(jnp.int32)
    rS = card_ranks[b0:, CARD_END - 1].astype(jnp.int32)
    sS = card_suits[b0:, CARD_END - 1].astype(jnp.int32)

    ctable = _build_table(rank_emb, suit_emb, street_emb)

    sc = pl.kernel(
        functools.partial(_sc_body, nchunks, n_tc),
        out_type=jax.ShapeDtypeStruct((n_rows, D_MODEL), jnp.float32),
        mesh=plsc.VectorSubcoreMesh(core_axis_name="c", subcore_axis_name="s"),
        scratch_types=[
            pltpu.VMEM((nchunks * CHUNK,), jnp.int32),
            pltpu.VMEM((nchunks * CHUNK,), jnp.int32),
            pltpu.VMEM((nchunks * CHUNK,), jnp.int32),
            pltpu.VMEM((nchunks * CHUNK,), jnp.int32),
            pltpu.VMEM((nchunks * CHUNK,), jnp.int32),
            pltpu.VMEM((NBUF, CHUNK, D_MODEL), jnp.float32),
            pltpu.SemaphoreType.DMA((NBUF,)),
            pltpu.SemaphoreType.DMA((NBUF,)),
        ],
    )
    y = sc(tokS, stS, rS, sS, ctable)

    nblk = n_tc // TC_ROWS
    blk3 = lambda a: a[:n_tc].reshape(nblk, TC_ROWS // 128, 128)
    stacked = jnp.concatenate([
        jnp.pad(rank_emb, ((0, 3), (0, 0))),
        jnp.pad(suit_emb, ((0, 4), (0, 0))),
        jnp.pad(street_emb, ((0, EMB_K - 28), (0, 0))),
    ], axis=0)
    def trunc16(x):
        u = lax.bitcast_convert_type(x, jnp.uint32)
        return lax.bitcast_convert_type(u & jnp.uint32(0xFFFF0000), jnp.float32)
    e1f = trunc16(stacked)
    r1 = stacked - e1f
    e2f = trunc16(r1)
    r2 = r1 - e2f
    e123 = jnp.concatenate([e1f.astype(jnp.bfloat16),
                            e2f.astype(jnp.bfloat16),
                            r2.astype(jnp.bfloat16)], axis=0)
    idx_spec = pl.BlockSpec((1, TC_ROWS // 128, 128), lambda i: (i, 0, 0))
    y = pl.pallas_call(
        _tc_main_body,
        grid=(nblk,),
        in_specs=[idx_spec, idx_spec, idx_spec, idx_spec,
                  pl.BlockSpec((3 * EMB_K, D_MODEL), lambda i: (0, 0)),
                  pl.BlockSpec(memory_space=pl.ANY)],
        out_specs=pl.BlockSpec((TC_ROWS, D_MODEL), lambda i: (i, 0)),
        out_shape=jax.ShapeDtypeStruct((n_rows, D_MODEL), jnp.float32),
        input_output_aliases={5: 0},
    )(blk3(tok7), blk3(st7), blk3(r7), blk3(s7), e123, y)

    return jnp.transpose(y.reshape(NCARD, B, D_MODEL), (1, 0, 2))
